# Initial kernel scaffold; baseline (speedup 1.0000x reference)
import jax
import jax.numpy as jnp
from jax.experimental import pallas as pl


def kernel(x, table, fc_w, fc_b):
    raise NotImplementedError("write your pallas kernel here")


# speedup vs baseline: 1.0946x; 1.0946x over previous
import functools

import jax
import jax.numpy as jnp
from jax import lax
from jax.experimental import pallas as pl
from jax.experimental.pallas import tpu as pltpu
from jax.experimental.pallas import tpu_sc as plsc


def _matvec_body(tbl_ref, w_ref, b_ref, s_ref):
    s_ref[...] = lax.dot_general(
        w_ref[...], tbl_ref[...],
        dimension_numbers=(((1,), (1,)), ((), ())),
        preferred_element_type=jnp.float32,
    ) + b_ref[0, 0]


def _scored_table(table, fc_w, fc_b):
    v, d = table.shape
    vb = 16384
    grid = -(-v // vb)
    s2 = pl.pallas_call(
        _matvec_body,
        grid=(grid,),
        in_specs=[
            pl.BlockSpec((vb, d), lambda i: (i, 0)),
            pl.BlockSpec((1, d), lambda i: (0, 0)),
            pl.BlockSpec((1, 1), lambda i: (0, 0)),
        ],
        out_specs=pl.BlockSpec((1, vb), lambda i: (0, i)),
        out_shape=jax.ShapeDtypeStruct((1, v), jnp.float32),
    )(table, fc_w, fc_b.reshape(1, 1))
    return s2.reshape(v)


def _transpose_body(x_ref, xt_ref):
    xt_ref[...] = x_ref[...].T


def _blockwise_transpose(x, rows_w):
    b, l = x.shape
    nblk = b // rows_w
    return pl.pallas_call(
        _transpose_body,
        grid=(nblk,),
        in_specs=[pl.BlockSpec((rows_w, l), lambda i: (i, 0))],
        out_specs=pl.BlockSpec((l, rows_w), lambda i: (i, 0)),
        out_shape=jax.ShapeDtypeStruct((nblk * l, rows_w), jnp.int32),
    )(x)


def _make_sc_pool(b: int, l: int):
    info = plsc.get_sparse_core_info()
    nc, ns = info.num_cores, info.num_subcores
    nw = nc * ns
    rows_w = b // nw
    n_idx = rows_w * l
    groups = rows_w // 16
    assert b % nw == 0 and rows_w % 16 == 0 and rows_w <= 128

    mesh = plsc.VectorSubcoreMesh(core_axis_name="c", subcore_axis_name="s")

    @functools.partial(
        pl.kernel,
        out_type=jax.ShapeDtypeStruct((b,), jnp.float32),
        mesh=mesh,
        scratch_types=[
            pltpu.VMEM((n_idx,), jnp.int32),
            pltpu.VMEM((n_idx,), jnp.float32),
            pltpu.VMEM((rows_w,), jnp.float32),
            pltpu.SemaphoreType.DMA,
        ],
    )
    def pool(xt_hbm, s_hbm, out_hbm, idx_v, val_v, out_v, sem):
        wid = lax.axis_index("s") * nc + lax.axis_index("c")
        base = wid * n_idx

        pltpu.sync_copy(xt_hbm.at[pl.ds(base, n_idx)], idx_v)

        def fire(c, _):
            pltpu.make_async_copy(
                s_hbm.at[idx_v.at[pl.ds(c * rows_w, rows_w)]],
                val_v.at[pl.ds(c * rows_w, rows_w)],
                sem,
            ).start()
            return 0

        lax.fori_loop(0, l, fire, 0)

        def drain(c, _):
            pltpu.make_async_copy(
                s_hbm.at[idx_v.at[pl.ds(c * rows_w, rows_w)]],
                val_v.at[pl.ds(c * rows_w, rows_w)],
                sem,
            ).wait()
            return 0

        lax.fori_loop(0, l, drain, 0)

        def body(c, accs):
            off = c * rows_w
            return tuple(
                accs[g] + val_v[pl.ds(off + g * 16, 16)] for g in range(groups)
            )

        accs = lax.fori_loop(
            0, l, body, tuple(jnp.zeros((16,), jnp.float32) for _ in range(groups))
        )

        inv_l = jnp.float32(1.0 / l)
        for g in range(groups):
            z = accs[g] * inv_l
            out_v[pl.ds(g * 16, 16)] = 1.0 / (1.0 + jnp.exp(-z))

        pltpu.sync_copy(out_v, out_hbm.at[pl.ds(wid * rows_w, rows_w)])

    return pool


def kernel(x, table, fc_w, fc_b):
    b, l = x.shape
    s = _scored_table(table, fc_w, fc_b)
    xt = _blockwise_transpose(x, b // 32)
    pool = _make_sc_pool(b, l)
    out = pool(xt.reshape(-1), s)
    return out.reshape(b, 1)


# speedup vs baseline: 1.1486x; 1.0493x over previous
import functools

import jax
import jax.numpy as jnp
from jax import lax
from jax.experimental import pallas as pl
from jax.experimental.pallas import tpu as pltpu
from jax.experimental.pallas import tpu_sc as plsc


def _matvec_body(t0, t1, t2, t3, w_ref, b_ref, s0, s1, s2, s3):
    w = w_ref[...]
    bias = b_ref[0, 0]
    for t_ref, s_ref in ((t0, s0), (t1, s1), (t2, s2), (t3, s3)):
        s_ref[0] = lax.dot_general(
            w, t_ref[...],
            dimension_numbers=(((1,), (1,)), ((), ())),
            preferred_element_type=jnp.float32,
        ) + bias


def _scored_table(table, fc_w, fc_b):
    v, d = table.shape
    vb = 10000
    nops = 4
    assert v % (nops * vb) == 0
    g = v // (nops * vb)
    outs = pl.pallas_call(
        _matvec_body,
        grid=(g,),
        in_specs=[
            pl.BlockSpec((vb, d), (lambda i, k=k: (nops * i + k, 0)))
            for k in range(nops)
        ] + [
            pl.BlockSpec((1, d), lambda i: (0, 0)),
            pl.BlockSpec((1, 1), lambda i: (0, 0)),
        ],
        out_specs=[
            pl.BlockSpec((1, 1, vb), lambda i: (i, 0, 0)) for _ in range(nops)
        ],
        out_shape=[
            jax.ShapeDtypeStruct((g, 1, vb), jnp.float32) for _ in range(nops)
        ],
    )(table, table, table, table, fc_w, fc_b.reshape(1, 1))
    return jnp.concatenate(outs, axis=1).reshape(v)


def _transpose_body(x_ref, xt_ref):
    xt_ref[...] = x_ref[...].T


def _blockwise_transpose(x, rows_w):
    b, l = x.shape
    nblk = b // rows_w
    return pl.pallas_call(
        _transpose_body,
        grid=(nblk,),
        in_specs=[pl.BlockSpec((rows_w, l), lambda i: (i, 0))],
        out_specs=pl.BlockSpec((l, rows_w), lambda i: (i, 0)),
        out_shape=jax.ShapeDtypeStruct((nblk * l, rows_w), jnp.int32),
    )(x)


def _make_sc_pool(b: int, l: int):
    info = plsc.get_sparse_core_info()
    nc, ns = info.num_cores, info.num_subcores
    nw = nc * ns
    rows_w = b // nw
    n_idx = rows_w * l
    groups = rows_w // 16
    assert b % nw == 0 and rows_w % 16 == 0 and rows_w <= 128

    mesh = plsc.VectorSubcoreMesh(core_axis_name="c", subcore_axis_name="s")

    @functools.partial(
        pl.kernel,
        out_type=jax.ShapeDtypeStruct((b,), jnp.float32),
        mesh=mesh,
        scratch_types=[
            pltpu.VMEM((n_idx,), jnp.int32),
            pltpu.VMEM((n_idx,), jnp.float32),
            pltpu.VMEM((rows_w,), jnp.float32),
            pltpu.SemaphoreType.DMA,
        ],
    )
    def pool(xt_hbm, s_hbm, out_hbm, idx_v, val_v, out_v, sem):
        wid = lax.axis_index("s") * nc + lax.axis_index("c")
        base = wid * n_idx

        pltpu.sync_copy(xt_hbm.at[pl.ds(base, n_idx)], idx_v)

        def fire(c, _):
            pltpu.make_async_copy(
                s_hbm.at[idx_v.at[pl.ds(c * rows_w, rows_w)]],
                val_v.at[pl.ds(c * rows_w, rows_w)],
                sem,
            ).start()
            return 0

        lax.fori_loop(0, l, fire, 0)

        def drain(c, _):
            pltpu.make_async_copy(
                s_hbm.at[idx_v.at[pl.ds(c * rows_w, rows_w)]],
                val_v.at[pl.ds(c * rows_w, rows_w)],
                sem,
            ).wait()
            return 0

        lax.fori_loop(0, l, drain, 0)

        def body(c, accs):
            off = c * rows_w
            return tuple(
                accs[g] + val_v[pl.ds(off + g * 16, 16)] for g in range(groups)
            )

        accs = lax.fori_loop(
            0, l, body, tuple(jnp.zeros((16,), jnp.float32) for _ in range(groups))
        )

        inv_l = jnp.float32(1.0 / l)
        for g in range(groups):
            z = accs[g] * inv_l
            out_v[pl.ds(g * 16, 16)] = 1.0 / (1.0 + jnp.exp(-z))

        pltpu.sync_copy(out_v, out_hbm.at[pl.ds(wid * rows_w, rows_w)])

    return pool


def kernel(x, table, fc_w, fc_b):
    b, l = x.shape
    s = _scored_table(table, fc_w, fc_b)
    xt = _blockwise_transpose(x, b // 32)
    pool = _make_sc_pool(b, l)
    out = pool(xt.reshape(-1), s)
    return out.reshape(b, 1)


# speedup vs baseline: 3.5279x; 3.0716x over previous
import functools

import jax
import jax.numpy as jnp
from jax import lax
from jax.experimental import pallas as pl
from jax.experimental.pallas import tpu as pltpu
from jax.experimental.pallas import tpu_sc as plsc


def _matvec_body(tt_ref, w_ref, b_ref, s_ref):
    s_ref[...] = lax.dot_general(
        w_ref[...], tt_ref[...],
        dimension_numbers=(((1,), (0,)), ((), ())),
        preferred_element_type=jnp.float32,
    ) + b_ref[0, 0]


def _scored_table(table, fc_w, fc_b):
    v, d = table.shape
    vb = 32768
    grid = -(-v // vb)
    s2 = pl.pallas_call(
        _matvec_body,
        grid=(grid,),
        in_specs=[
            pl.BlockSpec((d, vb), lambda i: (0, i)),
            pl.BlockSpec((1, d), lambda i: (0, 0)),
            pl.BlockSpec((1, 1), lambda i: (0, 0)),
        ],
        out_specs=pl.BlockSpec((1, vb), lambda i: (0, i)),
        out_shape=jax.ShapeDtypeStruct((1, v), jnp.float32),
    )(table.T, fc_w, fc_b.reshape(1, 1))
    return s2.reshape(v)


def _transpose_body(x_ref, xt_ref):
    xt_ref[...] = x_ref[...]


def _blockwise_transpose(x, rows_w):
    b, l = x.shape
    nblk = b // rows_w
    return pl.pallas_call(
        _transpose_body,
        grid=(nblk,),
        in_specs=[pl.BlockSpec((l, rows_w), lambda i: (0, i))],
        out_specs=pl.BlockSpec((l, rows_w), lambda i: (i, 0)),
        out_shape=jax.ShapeDtypeStruct((nblk * l, rows_w), jnp.int32),
    )(x.T)


def _make_sc_pool(b: int, l: int):
    info = plsc.get_sparse_core_info()
    nc, ns = info.num_cores, info.num_subcores
    nw = nc * ns
    rows_w = b // nw
    n_idx = rows_w * l
    groups = rows_w // 16
    assert b % nw == 0 and rows_w % 16 == 0 and rows_w <= 128

    mesh = plsc.VectorSubcoreMesh(core_axis_name="c", subcore_axis_name="s")

    @functools.partial(
        pl.kernel,
        out_type=jax.ShapeDtypeStruct((b,), jnp.float32),
        mesh=mesh,
        scratch_types=[
            pltpu.VMEM((n_idx,), jnp.int32),
            pltpu.VMEM((n_idx,), jnp.float32),
            pltpu.VMEM((rows_w,), jnp.float32),
            pltpu.SemaphoreType.DMA,
        ],
    )
    def pool(xt_hbm, s_hbm, out_hbm, idx_v, val_v, out_v, sem):
        wid = lax.axis_index("s") * nc + lax.axis_index("c")
        base = wid * n_idx

        pltpu.sync_copy(xt_hbm.at[pl.ds(base, n_idx)], idx_v)

        def fire(c, _):
            pltpu.make_async_copy(
                s_hbm.at[idx_v.at[pl.ds(c * rows_w, rows_w)]],
                val_v.at[pl.ds(c * rows_w, rows_w)],
                sem,
            ).start()
            return 0

        lax.fori_loop(0, l, fire, 0)

        def drain(c, _):
            pltpu.make_async_copy(
                s_hbm.at[idx_v.at[pl.ds(c * rows_w, rows_w)]],
                val_v.at[pl.ds(c * rows_w, rows_w)],
                sem,
            ).wait()
            return 0

        lax.fori_loop(0, l, drain, 0)

        def body(c, accs):
            off = c * rows_w
            return tuple(
                accs[g] + val_v[pl.ds(off + g * 16, 16)] for g in range(groups)
            )

        accs = lax.fori_loop(
            0, l, body, tuple(jnp.zeros((16,), jnp.float32) for _ in range(groups))
        )

        inv_l = jnp.float32(1.0 / l)
        for g in range(groups):
            z = accs[g] * inv_l
            out_v[pl.ds(g * 16, 16)] = 1.0 / (1.0 + jnp.exp(-z))

        pltpu.sync_copy(out_v, out_hbm.at[pl.ds(wid * rows_w, rows_w)])

    return pool


def kernel(x, table, fc_w, fc_b):
    b, l = x.shape
    s = _scored_table(table, fc_w, fc_b)
    xt = _blockwise_transpose(x, b // 32)
    pool = _make_sc_pool(b, l)
    out = pool(xt.reshape(-1), s)
    return out.reshape(b, 1)


# speedup vs baseline: 4.5804x; 1.2983x over previous
import functools

import jax
import jax.numpy as jnp
from jax import lax
from jax.experimental import pallas as pl
from jax.experimental.pallas import tpu as pltpu
from jax.experimental.pallas import tpu_sc as plsc


def _matvec_body(tt_ref, w_ref, b_ref, s_ref):
    s_ref[...] = (lax.dot_general(
        w_ref[...], tt_ref[...],
        dimension_numbers=(((1,), (0,)), ((), ())),
        preferred_element_type=jnp.float32,
    ) + b_ref[0, 0]).reshape(s_ref.shape)


def _scored_table(table, fc_w, fc_b):
    v, d = table.shape
    vb = 32768
    grid = -(-v // vb)
    s2 = pl.pallas_call(
        _matvec_body,
        grid=(grid,),
        in_specs=[
            pl.BlockSpec((d, vb), lambda i: (0, i)),
            pl.BlockSpec((1, d), lambda i: (0, 0)),
            pl.BlockSpec((1, 1), lambda i: (0, 0)),
        ],
        out_specs=pl.BlockSpec((vb,), lambda i: (i,)),
        out_shape=jax.ShapeDtypeStruct((v,), jnp.float32),
    )(table.T, fc_w, fc_b.reshape(1, 1))
    return s2


def _transpose_body(x_ref, xt_ref):
    xt_ref[...] = x_ref[...].reshape(xt_ref.shape)


def _blockwise_transpose(x, rows_w):
    b, l = x.shape
    nblk = b // rows_w
    return pl.pallas_call(
        _transpose_body,
        grid=(nblk,),
        in_specs=[pl.BlockSpec((l, rows_w), lambda i: (0, i))],
        out_specs=pl.BlockSpec((l * rows_w,), lambda i: (i,)),
        out_shape=jax.ShapeDtypeStruct((nblk * l * rows_w,), jnp.int32),
    )(x.T)


def _make_sc_pool(b: int, l: int):
    info = plsc.get_sparse_core_info()
    nc, ns = info.num_cores, info.num_subcores
    nw = nc * ns
    rows_w = b // nw
    n_idx = rows_w * l
    groups = rows_w // 16
    assert b % nw == 0 and rows_w % 16 == 0 and rows_w <= 128

    mesh = plsc.VectorSubcoreMesh(core_axis_name="c", subcore_axis_name="s")

    @functools.partial(
        pl.kernel,
        out_type=jax.ShapeDtypeStruct((b,), jnp.float32),
        mesh=mesh,
        scratch_types=[
            pltpu.VMEM((n_idx,), jnp.int32),
            pltpu.VMEM((n_idx,), jnp.float32),
            pltpu.VMEM((rows_w,), jnp.float32),
            pltpu.SemaphoreType.DMA,
        ],
    )
    def pool(xt_hbm, s_hbm, out_hbm, idx_v, val_v, out_v, sem):
        wid = lax.axis_index("s") * nc + lax.axis_index("c")
        base = wid * n_idx

        pltpu.sync_copy(xt_hbm.at[pl.ds(base, n_idx)], idx_v)

        def fire(c, _):
            pltpu.make_async_copy(
                s_hbm.at[idx_v.at[pl.ds(c * rows_w, rows_w)]],
                val_v.at[pl.ds(c * rows_w, rows_w)],
                sem,
            ).start()
            return 0

        lax.fori_loop(0, l, fire, 0)

        def body(c, accs):
            pltpu.make_async_copy(
                s_hbm.at[idx_v.at[pl.ds(c * rows_w, rows_w)]],
                val_v.at[pl.ds(c * rows_w, rows_w)],
                sem,
            ).wait()
            off = c * rows_w
            return tuple(
                accs[g] + val_v[pl.ds(off + g * 16, 16)] for g in range(groups)
            )

        accs = lax.fori_loop(
            0, l, body, tuple(jnp.zeros((16,), jnp.float32) for _ in range(groups))
        )

        inv_l = jnp.float32(1.0 / l)
        for g in range(groups):
            z = accs[g] * inv_l
            out_v[pl.ds(g * 16, 16)] = 1.0 / (1.0 + jnp.exp(-z))

        pltpu.sync_copy(out_v, out_hbm.at[pl.ds(wid * rows_w, rows_w)])

    return pool


def kernel(x, table, fc_w, fc_b):
    b, l = x.shape
    s = _scored_table(table, fc_w, fc_b)
    xt = _blockwise_transpose(x, b // 32)
    pool = _make_sc_pool(b, l)
    out = pool(xt, s)
    return out.reshape(b, 1)


# speedup vs baseline: 5.2400x; 1.1440x over previous
import functools

import jax
import jax.numpy as jnp
from jax import lax
from jax.experimental import pallas as pl
from jax.experimental.pallas import tpu as pltpu
from jax.experimental.pallas import tpu_sc as plsc


def _matvec_body(tt_ref, w_ref, b_ref, s_ref):
    s_ref[...] = (lax.dot_general(
        w_ref[...], tt_ref[...],
        dimension_numbers=(((1,), (0,)), ((), ())),
        preferred_element_type=jnp.float32,
    ) + b_ref[0, 0]).reshape(s_ref.shape)


def _scored_table(table, fc_w, fc_b):
    v, d = table.shape
    vb = 32768
    grid = -(-v // vb)
    s2 = pl.pallas_call(
        _matvec_body,
        grid=(grid,),
        in_specs=[
            pl.BlockSpec((d, vb), lambda i: (0, i)),
            pl.BlockSpec((1, d), lambda i: (0, 0)),
            pl.BlockSpec((1, 1), lambda i: (0, 0)),
        ],
        out_specs=pl.BlockSpec((vb,), lambda i: (i,)),
        out_shape=jax.ShapeDtypeStruct((v,), jnp.float32),
    )(table.T, fc_w, fc_b.reshape(1, 1))
    return s2


def _make_sc_pool(b: int, l: int):
    info = plsc.get_sparse_core_info()
    nc, ns = info.num_cores, info.num_subcores
    nw = nc * ns
    rows_w = b // nw
    n_idx = rows_w * l
    groups = rows_w // 16
    assert b % nw == 0 and rows_w % 16 == 0 and rows_w <= 128

    mesh = plsc.VectorSubcoreMesh(core_axis_name="c", subcore_axis_name="s")

    @functools.partial(
        pl.kernel,
        out_type=jax.ShapeDtypeStruct((b,), jnp.float32),
        mesh=mesh,
        scratch_types=[
            pltpu.VMEM((l, rows_w), jnp.int32),
            pltpu.VMEM((n_idx,), jnp.float32),
            pltpu.VMEM((rows_w,), jnp.float32),
            pltpu.SemaphoreType.DMA,
        ],
    )
    def pool(xt_hbm, s_hbm, out_hbm, idx_v, val_v, out_v, sem):
        wid = lax.axis_index("s") * nc + lax.axis_index("c")

        pltpu.sync_copy(xt_hbm.at[:, pl.ds(wid * rows_w, rows_w)], idx_v)

        def fire(c, _):
            pltpu.make_async_copy(
                s_hbm.at[idx_v.at[c]],
                val_v.at[pl.ds(c * rows_w, rows_w)],
                sem,
            ).start()
            return 0

        lax.fori_loop(0, l, fire, 0)

        def body(c, accs):
            pltpu.make_async_copy(
                s_hbm.at[idx_v.at[c]],
                val_v.at[pl.ds(c * rows_w, rows_w)],
                sem,
            ).wait()
            off = c * rows_w
            return tuple(
                accs[g] + val_v[pl.ds(off + g * 16, 16)] for g in range(groups)
            )

        accs = lax.fori_loop(
            0, l, body, tuple(jnp.zeros((16,), jnp.float32) for _ in range(groups))
        )

        inv_l = jnp.float32(1.0 / l)
        for g in range(groups):
            z = accs[g] * inv_l
            out_v[pl.ds(g * 16, 16)] = 1.0 / (1.0 + jnp.exp(-z))

        pltpu.sync_copy(out_v, out_hbm.at[pl.ds(wid * rows_w, rows_w)])

    return pool


def kernel(x, table, fc_w, fc_b):
    b, l = x.shape
    s = _scored_table(table, fc_w, fc_b)
    pool = _make_sc_pool(b, l)
    out = pool(x.T, s)
    return out.reshape(b, 1)
